# baseline (speedup 1.0000x reference)
import functools

import jax
import jax.numpy as jnp
from jax import lax
from jax.experimental import pallas as pl
from jax.experimental.pallas import tpu as pltpu
from jax.experimental.pallas import tpu_sc as plsc

N = 10000
E = 320000
D_IN = 128
H = 64
G = 2048

NC = 2
NS = 16
NW = NC * NS
C = 128
NCHUNKS = E // C
NCH_BASE = NCHUNKS // NW
XTRA = NCHUNKS % NW
RPT = 632
RPT_LAST = N - RPT * (NS - 1)
GPT = G // NS


def _chunk_start(wid):
    return wid * NCH_BASE + jnp.minimum(wid, XTRA)


def _load_idx(idx_hbm, idx_v, wid, sem):
    base = _chunk_start(wid) * C
    descs = []
    for k in range(NCH_BASE):
        descs.append(pltpu.async_copy(
            idx_hbm.at[pl.ds(base + k * C, C)], idx_v.at[k], sem))
    for d in descs:
        d.wait()

    @pl.when(wid < XTRA)
    def _():
        pltpu.async_copy(
            idx_hbm.at[pl.ds(base + NCH_BASE * C, C)], idx_v.at[NCH_BASE], sem
        ).wait()


def _tile_rows_copy(src, dst, s, add=False):
    @pl.when(s < NS - 1)
    def _():
        pltpu.sync_copy(src.at[pl.ds(s * RPT, RPT)],
                        dst.at[pl.ds(s * RPT, RPT)], add=add)

    @pl.when(s == NS - 1)
    def _():
        pltpu.sync_copy(src.at[pl.ds(RPT * (NS - 1), RPT_LAST)],
                        dst.at[pl.ds(RPT * (NS - 1), RPT_LAST)], add=add)

_MESH = plsc.VectorSubcoreMesh(
    core_axis_name="c", subcore_axis_name="s", num_cores=NC, num_subcores=NS
)


@functools.partial(
    pl.kernel,
    out_type=jax.ShapeDtypeStruct((NC, N, 16), jnp.float32),
    mesh=_MESH,
    compiler_params=pltpu.CompilerParams(use_tc_tiling_on_sc=False),
    scratch_types=[
        pltpu.VMEM((NCH_BASE + 1, C), jnp.int32),
        pltpu.VMEM((C, 16), jnp.float32),
        pltpu.VMEM_SHARED((N, 16), jnp.float32),
        pltpu.SemaphoreType.DMA,
    ],
)
def _deg_kernel(dst_hbm, ones_hbm, zeros_hbm, out_hbm, dst_v, ones_v, deg_sh, sem):
    c = lax.axis_index("c")
    s = lax.axis_index("s")
    wid = c * NS + s
    _load_idx(dst_hbm, dst_v, wid, sem)
    pltpu.sync_copy(ones_hbm, ones_v)
    _tile_rows_copy(zeros_hbm, deg_sh, s)
    plsc.subcore_barrier()
    descs = []
    for k in range(NCH_BASE):
        descs.append(pltpu.async_copy(ones_v, deg_sh.at[dst_v.at[k]], sem, add=True))
    for d in descs:
        d.wait()

    @pl.when(wid < XTRA)
    def _():
        pltpu.async_copy(ones_v, deg_sh.at[dst_v.at[NCH_BASE]], sem, add=True).wait()

    plsc.subcore_barrier()
    _tile_rows_copy(deg_sh, out_hbm.at[c], s)


NBUF = 8
LOOK = 6


def _msg_body(g_hbm, src_v, dst_v, part_sh, bufs, gsems, ssems, wid):
    gd = [None] * NCH_BASE
    sd = [None] * NCH_BASE
    for j in range(LOOK):
        gd[j] = pltpu.async_copy(g_hbm.at[src_v.at[j]], bufs[j % NBUF], gsems[j % NBUF])
    for k in range(NCH_BASE):
        b = k % NBUF
        gd[k].wait()
        sd[k] = pltpu.async_copy(bufs[b], part_sh.at[dst_v.at[k]], ssems[b], add=True)
        j = k + LOOK
        if j < NCH_BASE:
            if j - NBUF >= 0:
                sd[j - NBUF].wait()
            gd[j] = pltpu.async_copy(g_hbm.at[src_v.at[j]], bufs[j % NBUF], gsems[j % NBUF])
    for k in range(NCH_BASE - NBUF, NCH_BASE):
        sd[k].wait()

    @pl.when(wid < XTRA)
    def _():
        pltpu.async_copy(g_hbm.at[src_v.at[NCH_BASE]], bufs[0], gsems[0]).wait()
        pltpu.async_copy(bufs[0], part_sh.at[dst_v.at[NCH_BASE]], ssems[0],
                         add=True).wait()


_MSG_SCRATCH = (
    [
        pltpu.VMEM((NCH_BASE + 1, C), jnp.int32),
        pltpu.VMEM((NCH_BASE + 1, C), jnp.int32),
        pltpu.VMEM_SHARED((N, H), jnp.float32),
    ]
    + [pltpu.VMEM((C, H), jnp.float32)] * NBUF
    + [pltpu.SemaphoreType.DMA] * (2 * NBUF)
)


@functools.partial(
    pl.kernel,
    out_type=jax.ShapeDtypeStruct((NC, N, H), jnp.float32),
    mesh=_MESH,
    compiler_params=pltpu.CompilerParams(use_tc_tiling_on_sc=False),
    scratch_types=_MSG_SCRATCH,
)
def _msg1_kernel(g_hbm, src_hbm, dst_hbm, zeros_hbm, out_hbm,
                 src_v, dst_v, part_sh, *rest):
    bufs = rest[:NBUF]
    gsems = rest[NBUF:2 * NBUF]
    ssems = rest[2 * NBUF:3 * NBUF]
    c = lax.axis_index("c")
    s = lax.axis_index("s")
    wid = c * NS + s
    _load_idx(src_hbm, src_v, wid, gsems[0])
    _load_idx(dst_hbm, dst_v, wid, gsems[1])
    _tile_rows_copy(zeros_hbm, part_sh, s)
    plsc.subcore_barrier()
    _msg_body(g_hbm, src_v, dst_v, part_sh, bufs, gsems, ssems, wid)
    plsc.subcore_barrier()
    _tile_rows_copy(part_sh, out_hbm.at[c], s)


@functools.partial(
    pl.kernel,
    out_type=(
        jax.ShapeDtypeStruct((NC, G, H), jnp.float32),
        jax.ShapeDtypeStruct((G, H), jnp.float32),
        jax.ShapeDtypeStruct((G, 16), jnp.float32),
    ),
    mesh=_MESH,
    compiler_params=pltpu.CompilerParams(use_tc_tiling_on_sc=False),
    scratch_types=_MSG_SCRATCH + [
        pltpu.VMEM((GPT,), jnp.int32),
        pltpu.VMEM((GPT, 16), jnp.float32),
    ],
)
def _msg2_kernel(g_hbm, src_hbm, dst_hbm, zeros_hbm, gidx_hbm, dinv_hbm,
                 pg_hbm, g2g_hbm, dtg_hbm,
                 src_v, dst_v, part_sh, *rest):
    bufs = rest[:NBUF]
    gsems = rest[NBUF:2 * NBUF]
    ssems = rest[2 * NBUF:3 * NBUF]
    gidx_v, drow_v = rest[3 * NBUF:]
    c = lax.axis_index("c")
    s = lax.axis_index("s")
    wid = c * NS + s
    _load_idx(src_hbm, src_v, wid, gsems[0])
    _load_idx(dst_hbm, dst_v, wid, gsems[1])
    dg = pltpu.async_copy(gidx_hbm.at[pl.ds(s * GPT, GPT)], gidx_v, gsems[2])
    _tile_rows_copy(zeros_hbm, part_sh, s)
    dg.wait()
    plsc.subcore_barrier()
    _msg_body(g_hbm, src_v, dst_v, part_sh, bufs, gsems, ssems, wid)
    plsc.subcore_barrier()
    d1 = pltpu.async_copy(part_sh.at[gidx_v], bufs[1], gsems[0])

    @pl.when(c == 0)
    def _():
        pltpu.async_copy(g_hbm.at[gidx_v], bufs[2], gsems[1]).wait()
        pltpu.sync_copy(bufs[2], g2g_hbm.at[pl.ds(s * GPT, GPT)])

    @pl.when(c == 1)
    def _():
        pltpu.async_copy(dinv_hbm.at[gidx_v], drow_v, gsems[1]).wait()
        pltpu.sync_copy(drow_v, dtg_hbm.at[pl.ds(s * GPT, GPT)])

    d1.wait()
    pltpu.sync_copy(bufs[1], pg_hbm.at[c, pl.ds(s * GPT, GPT)])


_RB = 1000


def _mm1_body(x_ref, w1_ref, h1_ref):
    h1_ref[...] = lax.dot_general(x_ref[...], w1_ref[...],
                                  (((1,), (1,)), ((), ())),
                                  preferred_element_type=jnp.float32)


def _mm1(x, W1):
    return pl.pallas_call(
        _mm1_body,
        grid=(N // _RB,),
        in_specs=[
            pl.BlockSpec((_RB, D_IN), lambda i: (i, 0)),
            pl.BlockSpec((H, D_IN), lambda i: (0, 0)),
        ],
        out_specs=pl.BlockSpec((_RB, H), lambda i: (i, 0)),
        out_shape=jax.ShapeDtypeStruct((N, H), jnp.float32),
    )(x, W1)


def _scale1_body(h1_ref, degp_ref, g1_ref, dt_ref):
    deg16 = degp_ref[0] + degp_ref[1] + 1.0
    dinv16 = lax.rsqrt(deg16)
    g1_ref[...] = h1_ref[...] * dinv16[:, 0:1]
    dt_ref[...] = dinv16


def _scale1(h1, degp):
    return pl.pallas_call(
        _scale1_body,
        grid=(N // _RB,),
        in_specs=[
            pl.BlockSpec((_RB, H), lambda i: (i, 0)),
            pl.BlockSpec((NC, _RB, 16), lambda i: (0, i, 0)),
        ],
        out_specs=[
            pl.BlockSpec((_RB, H), lambda i: (i, 0)),
            pl.BlockSpec((_RB, 16), lambda i: (i, 0)),
        ],
        out_shape=[
            jax.ShapeDtypeStruct((N, H), jnp.float32),
            jax.ShapeDtypeStruct((N, 16), jnp.float32),
        ],
    )(h1, degp)


def _prep2_body(part_ref, g1_ref, dt_ref, w2_ref, sc1_ref, sh1_ref, g2_ref):
    dinv = dt_ref[...][:, 0:1]
    tot = (part_ref[0] + part_ref[1] + g1_ref[...]) * dinv
    m1 = jnp.maximum(tot * sc1_ref[...] + sh1_ref[...], 0.0)
    h2 = lax.dot_general(m1, w2_ref[...],
                         (((1,), (1,)), ((), ())),
                         preferred_element_type=jnp.float32)
    g2_ref[...] = h2 * dinv


def _prep2(part, g1, dt, W2, sc1, sh1):
    return pl.pallas_call(
        _prep2_body,
        grid=(N // _RB,),
        in_specs=[
            pl.BlockSpec((NC, _RB, H), lambda i: (0, i, 0)),
            pl.BlockSpec((_RB, H), lambda i: (i, 0)),
            pl.BlockSpec((_RB, 16), lambda i: (i, 0)),
            pl.BlockSpec((H, H), lambda i: (0, 0)),
            pl.BlockSpec((1, H), lambda i: (0, 0)),
            pl.BlockSpec((1, H), lambda i: (0, 0)),
        ],
        out_specs=pl.BlockSpec((_RB, H), lambda i: (i, 0)),
        out_shape=jax.ShapeDtypeStruct((N, H), jnp.float32),
    )(part, g1, dt, W2, sc1, sh1)


def _final_body(pg_ref, g2g_ref, dtg_ref, fcw_ref, fcb_ref, sc2_ref, sh2_ref, out_ref):
    dinv = dtg_ref[...][:, 0:1]
    tot = (pg_ref[0] + pg_ref[1] + g2g_ref[...]) * dinv
    m2 = jnp.maximum(tot * sc2_ref[...] + sh2_ref[...], 0.0)
    logits = lax.dot_general(m2, fcw_ref[...],
                             (((1,), (1,)), ((), ())),
                             preferred_element_type=jnp.float32) + fcb_ref[...]
    mx = jnp.max(logits, axis=1, keepdims=True)
    lse = jnp.log(jnp.sum(jnp.exp(logits - mx), axis=1, keepdims=True)) + mx
    out_ref[...] = logits - lse


def _final(pg, g2g, dtg, fc_W, fc_b, sc2, sh2):
    return pl.pallas_call(
        _final_body,
        grid=(1,),
        in_specs=[
            pl.BlockSpec((NC, G, H), lambda i: (0, 0, 0)),
            pl.BlockSpec((G, H), lambda i: (0, 0)),
            pl.BlockSpec((G, 16), lambda i: (0, 0)),
            pl.BlockSpec((2, H), lambda i: (0, 0)),
            pl.BlockSpec((1, 2), lambda i: (0, 0)),
            pl.BlockSpec((1, H), lambda i: (0, 0)),
            pl.BlockSpec((1, H), lambda i: (0, 0)),
        ],
        out_specs=pl.BlockSpec((G, 2), lambda i: (0, 0)),
        out_shape=jax.ShapeDtypeStruct((G, 2), jnp.float32),
    )(pg, g2g, dtg, fc_W, fc_b, sc2, sh2)


def kernel(x, edge_index, game_indices, W1, b1, bn1_gamma, bn1_beta, bn1_mean,
           bn1_var, W2, b2, bn2_gamma, bn2_beta, bn2_mean, bn2_var, fc_W, fc_b):
    src = edge_index[0]
    dst = edge_index[1]

    ones16 = jnp.ones((C, 16), jnp.float32)
    zeros16 = jnp.zeros((N, 16), jnp.float32)
    zerosH = jnp.zeros((N, H), jnp.float32)

    a1 = bn1_gamma * lax.rsqrt(bn1_var + 1e-5)
    sc1 = a1.reshape(1, H)
    sh1 = (b1 * a1 + bn1_beta - bn1_mean * a1).reshape(1, H)
    a2 = bn2_gamma * lax.rsqrt(bn2_var + 1e-5)
    sc2 = a2.reshape(1, H)
    sh2 = (b2 * a2 + bn2_beta - bn2_mean * a2).reshape(1, H)

    degp = _deg_kernel(dst, ones16, zeros16)
    h1 = _mm1(x, W1)
    g1, dt = _scale1(h1, degp)
    part1 = _msg1_kernel(g1, src, dst, zerosH)
    g2 = _prep2(part1, g1, dt, W2, sc1, sh1)
    pg, g2g, dtg = _msg2_kernel(g2, src, dst, zerosH, game_indices, dt)
    return _final(pg, g2g, dtg, fc_W, fc_b.reshape(1, 2), sc2, sh2)


# speedup vs baseline: 1.0160x; 1.0160x over previous
import functools

import jax
import jax.numpy as jnp
from jax import lax
from jax.experimental import pallas as pl
from jax.experimental.pallas import tpu as pltpu
from jax.experimental.pallas import tpu_sc as plsc

N = 10000
E = 320000
D_IN = 128
H = 64
G = 2048

NC = 2
NS = 16
NW = NC * NS
C = 128
NCHUNKS = E // C
NCH_BASE = NCHUNKS // NW
XTRA = NCHUNKS % NW
RPT = 632
RPT_LAST = N - RPT * (NS - 1)
GPT = G // NS


def _chunk_start(wid):
    return wid * NCH_BASE + jnp.minimum(wid, XTRA)


def _load_idx(idx_hbm, idx_v, wid, sem):
    @pl.when(wid < XTRA)
    def _():
        pltpu.async_copy(
            idx_hbm.at[pl.ds(_chunk_start(wid), NCH_BASE + 1)], idx_v, sem
        ).wait()

    @pl.when(wid >= XTRA)
    def _():
        pltpu.async_copy(
            idx_hbm.at[pl.ds(_chunk_start(wid), NCH_BASE)],
            idx_v.at[pl.ds(0, NCH_BASE)], sem
        ).wait()


def _tile_rows_copy(src, dst, s, add=False):
    @pl.when(s < NS - 1)
    def _():
        pltpu.sync_copy(src.at[pl.ds(s * RPT, RPT)],
                        dst.at[pl.ds(s * RPT, RPT)], add=add)

    @pl.when(s == NS - 1)
    def _():
        pltpu.sync_copy(src.at[pl.ds(RPT * (NS - 1), RPT_LAST)],
                        dst.at[pl.ds(RPT * (NS - 1), RPT_LAST)], add=add)

_MESH = plsc.VectorSubcoreMesh(
    core_axis_name="c", subcore_axis_name="s", num_cores=NC, num_subcores=NS
)


@functools.partial(
    pl.kernel,
    out_type=jax.ShapeDtypeStruct((NC, N, 16), jnp.float32),
    mesh=_MESH,
    compiler_params=pltpu.CompilerParams(use_tc_tiling_on_sc=False),
    scratch_types=[
        pltpu.VMEM((NCH_BASE + 1, C), jnp.int32),
        pltpu.VMEM((C, 16), jnp.float32),
        pltpu.VMEM_SHARED((N, 16), jnp.float32),
        pltpu.SemaphoreType.DMA,
    ],
)
def _deg_kernel(dst_hbm, ones_hbm, zeros_hbm, out_hbm, dst_v, ones_v, deg_sh, sem):
    c = lax.axis_index("c")
    s = lax.axis_index("s")
    wid = c * NS + s
    _load_idx(dst_hbm, dst_v, wid, sem)
    pltpu.sync_copy(ones_hbm, ones_v)
    _tile_rows_copy(zeros_hbm, deg_sh, s)
    plsc.subcore_barrier()
    descs = []
    for k in range(NCH_BASE):
        descs.append(pltpu.async_copy(ones_v, deg_sh.at[dst_v.at[k]], sem, add=True))
    for d in descs:
        d.wait()

    @pl.when(wid < XTRA)
    def _():
        pltpu.async_copy(ones_v, deg_sh.at[dst_v.at[NCH_BASE]], sem, add=True).wait()

    plsc.subcore_barrier()
    _tile_rows_copy(deg_sh, out_hbm.at[c], s)


NBUF = 8
LOOK = 6


def _msg_body(g_hbm, src_v, dst_v, part_sh, bufs, gsems, ssems, wid):
    gd = [None] * NCH_BASE
    sd = [None] * NCH_BASE
    for j in range(LOOK):
        gd[j] = pltpu.async_copy(g_hbm.at[src_v.at[j]], bufs[j % NBUF], gsems[j % NBUF])
    for k in range(NCH_BASE):
        b = k % NBUF
        gd[k].wait()
        sd[k] = pltpu.async_copy(bufs[b], part_sh.at[dst_v.at[k]], ssems[b], add=True)
        j = k + LOOK
        if j < NCH_BASE:
            if j - NBUF >= 0:
                sd[j - NBUF].wait()
            gd[j] = pltpu.async_copy(g_hbm.at[src_v.at[j]], bufs[j % NBUF], gsems[j % NBUF])
    for k in range(NCH_BASE - NBUF, NCH_BASE):
        sd[k].wait()

    @pl.when(wid < XTRA)
    def _():
        pltpu.async_copy(g_hbm.at[src_v.at[NCH_BASE]], bufs[0], gsems[0]).wait()
        pltpu.async_copy(bufs[0], part_sh.at[dst_v.at[NCH_BASE]], ssems[0],
                         add=True).wait()


_MSG_SCRATCH = (
    [
        pltpu.VMEM((NCH_BASE + 1, C), jnp.int32),
        pltpu.VMEM((NCH_BASE + 1, C), jnp.int32),
        pltpu.VMEM_SHARED((N, H), jnp.float32),
    ]
    + [pltpu.VMEM((C, H), jnp.float32)] * NBUF
    + [pltpu.SemaphoreType.DMA] * (2 * NBUF)
)


@functools.partial(
    pl.kernel,
    out_type=jax.ShapeDtypeStruct((NC, N, H), jnp.float32),
    mesh=_MESH,
    compiler_params=pltpu.CompilerParams(use_tc_tiling_on_sc=False),
    scratch_types=_MSG_SCRATCH,
)
def _msg1_kernel(g_hbm, src_hbm, dst_hbm, zeros_hbm, out_hbm,
                 src_v, dst_v, part_sh, *rest):
    bufs = rest[:NBUF]
    gsems = rest[NBUF:2 * NBUF]
    ssems = rest[2 * NBUF:3 * NBUF]
    c = lax.axis_index("c")
    s = lax.axis_index("s")
    wid = c * NS + s
    _load_idx(src_hbm, src_v, wid, gsems[0])
    _load_idx(dst_hbm, dst_v, wid, gsems[1])
    _tile_rows_copy(zeros_hbm, part_sh, s)
    plsc.subcore_barrier()
    _msg_body(g_hbm, src_v, dst_v, part_sh, bufs, gsems, ssems, wid)
    plsc.subcore_barrier()
    _tile_rows_copy(part_sh, out_hbm.at[c], s)


@functools.partial(
    pl.kernel,
    out_type=(
        jax.ShapeDtypeStruct((NC, G, H), jnp.float32),
        jax.ShapeDtypeStruct((G, H), jnp.float32),
        jax.ShapeDtypeStruct((G, 16), jnp.float32),
    ),
    mesh=_MESH,
    compiler_params=pltpu.CompilerParams(use_tc_tiling_on_sc=False),
    scratch_types=_MSG_SCRATCH + [
        pltpu.VMEM((GPT,), jnp.int32),
        pltpu.VMEM((GPT, 16), jnp.float32),
    ],
)
def _msg2_kernel(g_hbm, src_hbm, dst_hbm, zeros_hbm, gidx_hbm, dinv_hbm,
                 pg_hbm, g2g_hbm, dtg_hbm,
                 src_v, dst_v, part_sh, *rest):
    bufs = rest[:NBUF]
    gsems = rest[NBUF:2 * NBUF]
    ssems = rest[2 * NBUF:3 * NBUF]
    gidx_v, drow_v = rest[3 * NBUF:]
    c = lax.axis_index("c")
    s = lax.axis_index("s")
    wid = c * NS + s
    _load_idx(src_hbm, src_v, wid, gsems[0])
    _load_idx(dst_hbm, dst_v, wid, gsems[1])
    dg = pltpu.async_copy(gidx_hbm.at[pl.ds(s * GPT, GPT)], gidx_v, gsems[2])
    _tile_rows_copy(zeros_hbm, part_sh, s)
    dg.wait()
    plsc.subcore_barrier()
    _msg_body(g_hbm, src_v, dst_v, part_sh, bufs, gsems, ssems, wid)
    plsc.subcore_barrier()
    d1 = pltpu.async_copy(part_sh.at[gidx_v], bufs[1], gsems[0])

    @pl.when(c == 0)
    def _():
        pltpu.async_copy(g_hbm.at[gidx_v], bufs[2], gsems[1]).wait()
        pltpu.sync_copy(bufs[2], g2g_hbm.at[pl.ds(s * GPT, GPT)])

    @pl.when(c == 1)
    def _():
        pltpu.async_copy(dinv_hbm.at[gidx_v], drow_v, gsems[1]).wait()
        pltpu.sync_copy(drow_v, dtg_hbm.at[pl.ds(s * GPT, GPT)])

    d1.wait()
    pltpu.sync_copy(bufs[1], pg_hbm.at[c, pl.ds(s * GPT, GPT)])


_RB = 1000


def _mm1_body(x_ref, w1_ref, h1_ref):
    h1_ref[...] = lax.dot_general(x_ref[...], w1_ref[...],
                                  (((1,), (1,)), ((), ())),
                                  preferred_element_type=jnp.float32)


def _mm1(x, W1):
    return pl.pallas_call(
        _mm1_body,
        grid=(N // _RB,),
        in_specs=[
            pl.BlockSpec((_RB, D_IN), lambda i: (i, 0)),
            pl.BlockSpec((H, D_IN), lambda i: (0, 0)),
        ],
        out_specs=pl.BlockSpec((_RB, H), lambda i: (i, 0)),
        out_shape=jax.ShapeDtypeStruct((N, H), jnp.float32),
    )(x, W1)


def _scale1_body(h1_ref, degp_ref, g1_ref, dt_ref):
    deg16 = degp_ref[0] + degp_ref[1] + 1.0
    dinv16 = lax.rsqrt(deg16)
    g1_ref[...] = h1_ref[...] * dinv16[:, 0:1]
    dt_ref[...] = dinv16


def _scale1(h1, degp):
    return pl.pallas_call(
        _scale1_body,
        grid=(N // _RB,),
        in_specs=[
            pl.BlockSpec((_RB, H), lambda i: (i, 0)),
            pl.BlockSpec((NC, _RB, 16), lambda i: (0, i, 0)),
        ],
        out_specs=[
            pl.BlockSpec((_RB, H), lambda i: (i, 0)),
            pl.BlockSpec((_RB, 16), lambda i: (i, 0)),
        ],
        out_shape=[
            jax.ShapeDtypeStruct((N, H), jnp.float32),
            jax.ShapeDtypeStruct((N, 16), jnp.float32),
        ],
    )(h1, degp)


def _prep2_body(part_ref, g1_ref, dt_ref, w2_ref, sc1_ref, sh1_ref, g2_ref):
    dinv = dt_ref[...][:, 0:1]
    tot = (part_ref[0] + part_ref[1] + g1_ref[...]) * dinv
    m1 = jnp.maximum(tot * sc1_ref[...] + sh1_ref[...], 0.0)
    h2 = lax.dot_general(m1, w2_ref[...],
                         (((1,), (1,)), ((), ())),
                         preferred_element_type=jnp.float32)
    g2_ref[...] = h2 * dinv


def _prep2(part, g1, dt, W2, sc1, sh1):
    return pl.pallas_call(
        _prep2_body,
        grid=(N // _RB,),
        in_specs=[
            pl.BlockSpec((NC, _RB, H), lambda i: (0, i, 0)),
            pl.BlockSpec((_RB, H), lambda i: (i, 0)),
            pl.BlockSpec((_RB, 16), lambda i: (i, 0)),
            pl.BlockSpec((H, H), lambda i: (0, 0)),
            pl.BlockSpec((1, H), lambda i: (0, 0)),
            pl.BlockSpec((1, H), lambda i: (0, 0)),
        ],
        out_specs=pl.BlockSpec((_RB, H), lambda i: (i, 0)),
        out_shape=jax.ShapeDtypeStruct((N, H), jnp.float32),
    )(part, g1, dt, W2, sc1, sh1)


def _final_body(pg_ref, g2g_ref, dtg_ref, fcw_ref, fcb_ref, sc2_ref, sh2_ref, out_ref):
    dinv = dtg_ref[...][:, 0:1]
    tot = (pg_ref[0] + pg_ref[1] + g2g_ref[...]) * dinv
    m2 = jnp.maximum(tot * sc2_ref[...] + sh2_ref[...], 0.0)
    logits = lax.dot_general(m2, fcw_ref[...],
                             (((1,), (1,)), ((), ())),
                             preferred_element_type=jnp.float32) + fcb_ref[...]
    mx = jnp.max(logits, axis=1, keepdims=True)
    lse = jnp.log(jnp.sum(jnp.exp(logits - mx), axis=1, keepdims=True)) + mx
    out_ref[...] = logits - lse


def _final(pg, g2g, dtg, fc_W, fc_b, sc2, sh2):
    return pl.pallas_call(
        _final_body,
        grid=(1,),
        in_specs=[
            pl.BlockSpec((NC, G, H), lambda i: (0, 0, 0)),
            pl.BlockSpec((G, H), lambda i: (0, 0)),
            pl.BlockSpec((G, 16), lambda i: (0, 0)),
            pl.BlockSpec((2, H), lambda i: (0, 0)),
            pl.BlockSpec((1, 2), lambda i: (0, 0)),
            pl.BlockSpec((1, H), lambda i: (0, 0)),
            pl.BlockSpec((1, H), lambda i: (0, 0)),
        ],
        out_specs=pl.BlockSpec((G, 2), lambda i: (0, 0)),
        out_shape=jax.ShapeDtypeStruct((G, 2), jnp.float32),
    )(pg, g2g, dtg, fc_W, fc_b, sc2, sh2)


def kernel(x, edge_index, game_indices, W1, b1, bn1_gamma, bn1_beta, bn1_mean,
           bn1_var, W2, b2, bn2_gamma, bn2_beta, bn2_mean, bn2_var, fc_W, fc_b):
    src = edge_index[0].reshape(NCHUNKS, C)
    dst = edge_index[1].reshape(NCHUNKS, C)

    ones16 = jnp.ones((C, 16), jnp.float32)
    zeros16 = jnp.zeros((N, 16), jnp.float32)
    zerosH = jnp.zeros((N, H), jnp.float32)

    a1 = bn1_gamma * lax.rsqrt(bn1_var + 1e-5)
    sc1 = a1.reshape(1, H)
    sh1 = (b1 * a1 + bn1_beta - bn1_mean * a1).reshape(1, H)
    a2 = bn2_gamma * lax.rsqrt(bn2_var + 1e-5)
    sc2 = a2.reshape(1, H)
    sh2 = (b2 * a2 + bn2_beta - bn2_mean * a2).reshape(1, H)

    degp = _deg_kernel(dst, ones16, zeros16)
    h1 = _mm1(x, W1)
    g1, dt = _scale1(h1, degp)
    part1 = _msg1_kernel(g1, src, dst, zerosH)
    g2 = _prep2(part1, g1, dt, W2, sc1, sh1)
    pg, g2g, dtg = _msg2_kernel(g2, src, dst, zerosH, game_indices, dt)
    return _final(pg, g2g, dtg, fc_W, fc_b.reshape(1, 2), sc2, sh2)
